# Initial kernel scaffold; baseline (speedup 1.0000x reference)
import functools

import jax
import jax.numpy as jnp
from jax import lax
from jax.experimental import pallas as pl
from jax.experimental.pallas import tpu as pltpu
from jax.experimental.pallas import tpu_sc as plsc

N = 10000
E = 320000
F_IN = 128
F_OUT = 128

NC = 2
NS = 16
NW = NC * NS
L = 16

N_PAD = 10240
E_PER_W = E // NW
KA = 2000
KC = 80
ROWS_PER_TILE = N_PAD // NS
BLK = 256

_mesh = plsc.VectorSubcoreMesh(core_axis_name="c", subcore_axis_name="s")


@functools.partial(
    pl.kernel,
    out_type=(
        jax.ShapeDtypeStruct((NW, N_PAD), jnp.float32),
        jax.ShapeDtypeStruct((E,), jnp.float32),
    ),
    mesh=_mesh,
    scratch_types=[
        pltpu.VMEM((N_PAD,), jnp.float32),
        pltpu.VMEM((KA,), jnp.float32),
        pltpu.VMEM((KA,), jnp.int32),
        pltpu.VMEM((KA,), jnp.float32),
        pltpu.SemaphoreType.DMA,
    ],
)
def _deg_kernel(ew_hbm, dst_hbm, deg_out, sig_out, deg_v, ew_v, dst_v, sig_v, sem):
    w = lax.axis_index("c") * NS + lax.axis_index("s")
    base = w * E_PER_W
    zero16 = jnp.zeros((L,), jnp.float32)

    @pl.loop(0, N_PAD, step=L)
    def _(i):
        deg_v[pl.ds(i, L)] = zero16

    @pl.loop(0, E_PER_W, step=KA)
    def _(off):
        pltpu.async_copy(ew_hbm.at[pl.ds(base + off, KA)], ew_v, sem).wait()
        pltpu.async_copy(dst_hbm.at[pl.ds(base + off, KA)], dst_v, sem).wait()

        @pl.loop(0, KA, step=L)
        def _(j):
            wv = ew_v[pl.ds(j, L)]
            s = 1.0 / (1.0 + jnp.exp(-wv))
            sig_v[pl.ds(j, L)] = s
            di = dst_v[pl.ds(j, L)]
            plsc.addupdate_scatter(deg_v, [di], s)

        pltpu.async_copy(sig_v, sig_out.at[pl.ds(base + off, KA)], sem).wait()

    pltpu.async_copy(deg_v, deg_out.at[w], sem).wait()


@functools.partial(
    pl.pallas_call,
    grid=(N_PAD // BLK,),
    in_specs=[
        pl.BlockSpec((BLK, F_IN), lambda i: (i, 0)),
        pl.BlockSpec((F_IN, F_OUT), lambda i: (0, 0)),
        pl.BlockSpec((NW, BLK), lambda i: (0, i)),
    ],
    out_specs=pl.BlockSpec((BLK, F_OUT), lambda i: (i, 0)),
    out_shape=jax.ShapeDtypeStruct((N_PAD, F_OUT), jnp.float32),
)
def _h2_kernel(x_ref, w_ref, deg_ref, h2_ref):
    deg = 1.0 + jnp.sum(deg_ref[...], axis=0)
    dinv = lax.rsqrt(deg)
    h = jnp.dot(x_ref[...], w_ref[...], preferred_element_type=jnp.float32,
                precision=lax.Precision.HIGHEST)
    h2_ref[...] = h * dinv[:, None]


@functools.partial(
    pl.kernel,
    out_type=jax.ShapeDtypeStruct((NC, N_PAD, F_OUT), jnp.float32),
    mesh=_mesh,
    scratch_types=[
        pltpu.VMEM_SHARED((N_PAD, F_OUT), jnp.float32),
        pltpu.VMEM((KC, F_OUT), jnp.float32),
        pltpu.VMEM((KC,), jnp.int32),
        pltpu.VMEM((KC,), jnp.int32),
        pltpu.SMEM((KC,), jnp.float32),
        pltpu.SemaphoreType.DMA,
    ],
)
def _agg_kernel(h2_hbm, src_hbm, dst_hbm, sig_hbm, zero_hbm, out_hbm,
                acc_sh, rows_v, src_v, dst_v, sig_s, sem):
    c = lax.axis_index("c")
    s = lax.axis_index("s")
    w = c * NS + s
    base = w * E_PER_W
    row0 = s * ROWS_PER_TILE

    pltpu.async_copy(zero_hbm.at[pl.ds(row0, ROWS_PER_TILE)],
                     acc_sh.at[pl.ds(row0, ROWS_PER_TILE)], sem).wait()
    plsc.subcore_barrier()

    @pl.loop(0, E_PER_W, step=KC)
    def _(off):
        pltpu.async_copy(src_hbm.at[pl.ds(base + off, KC)], src_v, sem).wait()
        pltpu.async_copy(dst_hbm.at[pl.ds(base + off, KC)], dst_v, sem).wait()
        pltpu.async_copy(sig_hbm.at[pl.ds(base + off, KC)], sig_s, sem).wait()
        pltpu.async_copy(h2_hbm.at[src_v], rows_v, sem).wait()

        @pl.loop(0, KC)
        def _(k):
            sc = sig_s[k]
            for ccol in range(F_OUT // L):
                sl = pl.ds(ccol * L, L)
                rows_v[k, sl] = rows_v[k, sl] * sc

        pltpu.sync_copy(rows_v, acc_sh.at[dst_v], add=True)

    plsc.subcore_barrier()
    pltpu.async_copy(acc_sh.at[pl.ds(row0, ROWS_PER_TILE)],
                     out_hbm.at[c, pl.ds(row0, ROWS_PER_TILE)], sem).wait()


@functools.partial(
    pl.pallas_call,
    grid=(N_PAD // BLK,),
    in_specs=[
        pl.BlockSpec((1, BLK, F_OUT), lambda i: (0, i, 0)),
        pl.BlockSpec((1, BLK, F_OUT), lambda i: (1, i, 0)),
        pl.BlockSpec((BLK, F_OUT), lambda i: (i, 0)),
        pl.BlockSpec((NW, BLK), lambda i: (0, i)),
        pl.BlockSpec((1, F_OUT), lambda i: (0, 0)),
    ],
    out_specs=pl.BlockSpec((BLK, F_OUT), lambda i: (i, 0)),
    out_shape=jax.ShapeDtypeStruct((N_PAD, F_OUT), jnp.float32),
)
def _out_kernel(p0_ref, p1_ref, h2_ref, deg_ref, b_ref, o_ref):
    deg = 1.0 + jnp.sum(deg_ref[...], axis=0)
    dinv = lax.rsqrt(deg)
    o_ref[...] = (p0_ref[0] + p1_ref[0] + h2_ref[...]) * dinv[:, None] + b_ref[...]


def kernel(x, edge_index, edge_weight, W, b):
    assert x.shape == (N, F_IN) and edge_index.shape == (2, E)
    src = edge_index[0]
    dst = edge_index[1]
    x_pad = jnp.pad(x, ((0, N_PAD - N), (0, 0)))
    zeros = jnp.zeros((N_PAD, F_OUT), jnp.float32)

    deg_parts, sig = _deg_kernel(edge_weight, dst)
    h2 = _h2_kernel(x_pad, W, deg_parts)
    parts = _agg_kernel(h2, src, dst, sig, zeros)
    out_pad = _out_kernel(parts, parts, h2, deg_parts, b.reshape(1, F_OUT))
    return out_pad[:N]


# speedup vs baseline: 13.5186x; 13.5186x over previous
import dataclasses
import functools

import jax
import jax.numpy as jnp
from jax import lax
from jax.experimental import pallas as pl
from jax.experimental.pallas import tpu as pltpu
from jax.experimental.pallas import tpu_sc as plsc

N = 10000
E = 320000
F_IN = 128
F_OUT = 128

NC = 2
NS = 16
NW = NC * NS
L = 16

N_PAD = 10240
E_PER_W = E // NW
KA = 2000
KC = 80
ROWS_PER_TILE = N_PAD // NS
BLK = 256

_mesh = plsc.VectorSubcoreMesh(core_axis_name="c", subcore_axis_name="s")

_sc_params = pltpu.CompilerParams()
if "needs_layout_passes" in pltpu.CompilerParams.__dataclass_fields__:
    _sc_params = dataclasses.replace(_sc_params, needs_layout_passes=False)


@functools.partial(
    pl.kernel,
    out_type=(
        jax.ShapeDtypeStruct((NW, N_PAD), jnp.float32),
        jax.ShapeDtypeStruct((E,), jnp.float32),
    ),
    mesh=_mesh,
    scratch_types=[
        pltpu.VMEM((N_PAD,), jnp.float32),
        pltpu.VMEM((KA,), jnp.float32),
        pltpu.VMEM((KA,), jnp.int32),
        pltpu.VMEM((KA,), jnp.float32),
        pltpu.SemaphoreType.DMA,
    ],
    compiler_params=_sc_params,
)
def _deg_kernel(ew_hbm, dst_hbm, deg_out, sig_out, deg_v, ew_v, dst_v, sig_v, sem):
    w = lax.axis_index("c") * NS + lax.axis_index("s")
    base = w * E_PER_W
    zero16 = jnp.zeros((L,), jnp.float32)

    @pl.loop(0, N_PAD, step=L)
    def _(i):
        deg_v[pl.ds(i, L)] = zero16

    @pl.loop(0, E_PER_W, step=KA)
    def _(off):
        pltpu.async_copy(ew_hbm.at[pl.ds(base + off, KA)], ew_v, sem).wait()
        pltpu.async_copy(dst_hbm.at[pl.ds(base + off, KA)], dst_v, sem).wait()

        @pl.loop(0, KA, step=L)
        def _(j):
            wv = ew_v[pl.ds(j, L)]
            s = 1.0 / (1.0 + jnp.exp(-wv))
            sig_v[pl.ds(j, L)] = s
            di = dst_v[pl.ds(j, L)]
            plsc.addupdate_scatter(deg_v, [di], s)

        pltpu.async_copy(sig_v, sig_out.at[pl.ds(base + off, KA)], sem).wait()

    pltpu.async_copy(deg_v, deg_out.at[w], sem).wait()


@functools.partial(
    pl.pallas_call,
    grid=(N_PAD // BLK,),
    in_specs=[
        pl.BlockSpec((BLK, F_IN), lambda i: (i, 0)),
        pl.BlockSpec((F_IN, F_OUT), lambda i: (0, 0)),
        pl.BlockSpec((NW, BLK), lambda i: (0, i)),
    ],
    out_specs=pl.BlockSpec((BLK, F_OUT), lambda i: (i, 0)),
    out_shape=jax.ShapeDtypeStruct((N_PAD, F_OUT), jnp.float32),
)
def _h2_kernel(x_ref, w_ref, deg_ref, h2_ref):
    deg = 1.0 + jnp.sum(deg_ref[...], axis=0)
    dinv = lax.rsqrt(deg)
    h = jnp.dot(x_ref[...], w_ref[...], preferred_element_type=jnp.float32,
                precision=lax.Precision.HIGHEST)
    h2_ref[...] = h * dinv[:, None]


@functools.partial(
    pl.kernel,
    out_type=jax.ShapeDtypeStruct((NC, N_PAD, F_OUT), jnp.float32),
    mesh=_mesh,
    scratch_types=[
        pltpu.VMEM_SHARED((N_PAD, F_OUT), jnp.float32),
        pltpu.VMEM((KC, F_OUT), jnp.float32),
        pltpu.VMEM((KC,), jnp.int32),
        pltpu.VMEM((KC,), jnp.int32),
        pltpu.VMEM((KC,), jnp.float32),
        pltpu.SemaphoreType.DMA,
    ],
    compiler_params=_sc_params,
)
def _agg_kernel(h2_hbm, src_hbm, dst_hbm, sig_hbm, zero_hbm, out_hbm,
                acc_sh, rows_v, src_v, dst_v, sig_v, sem):
    c = lax.axis_index("c")
    s = lax.axis_index("s")
    w = c * NS + s
    base = w * E_PER_W
    row0 = s * ROWS_PER_TILE

    pltpu.async_copy(zero_hbm.at[pl.ds(row0, ROWS_PER_TILE)],
                     acc_sh.at[pl.ds(row0, ROWS_PER_TILE)], sem).wait()
    plsc.subcore_barrier()

    @pl.loop(0, E_PER_W, step=KC)
    def _(off):
        pltpu.async_copy(src_hbm.at[pl.ds(base + off, KC)], src_v, sem).wait()
        pltpu.async_copy(dst_hbm.at[pl.ds(base + off, KC)], dst_v, sem).wait()
        pltpu.async_copy(sig_hbm.at[pl.ds(base + off, KC)], sig_v, sem).wait()
        pltpu.async_copy(h2_hbm.at[src_v], rows_v, sem).wait()

        @pl.loop(0, KC)
        def _(k):
            kv = jnp.broadcast_to(k, (L,)).astype(jnp.int32)
            sv = plsc.load_gather(sig_v, [kv])
            for ccol in range(F_OUT // L):
                sl = pl.ds(ccol * L, L)
                rows_v[k, sl] = rows_v[k, sl] * sv

        pltpu.sync_copy(rows_v, acc_sh.at[dst_v], add=True)

    plsc.subcore_barrier()
    pltpu.async_copy(acc_sh.at[pl.ds(row0, ROWS_PER_TILE)],
                     out_hbm.at[c, pl.ds(row0, ROWS_PER_TILE)], sem).wait()


@functools.partial(
    pl.pallas_call,
    grid=(N_PAD // BLK,),
    in_specs=[
        pl.BlockSpec((1, BLK, F_OUT), lambda i: (0, i, 0)),
        pl.BlockSpec((1, BLK, F_OUT), lambda i: (1, i, 0)),
        pl.BlockSpec((BLK, F_OUT), lambda i: (i, 0)),
        pl.BlockSpec((NW, BLK), lambda i: (0, i)),
        pl.BlockSpec((1, F_OUT), lambda i: (0, 0)),
    ],
    out_specs=pl.BlockSpec((BLK, F_OUT), lambda i: (i, 0)),
    out_shape=jax.ShapeDtypeStruct((N_PAD, F_OUT), jnp.float32),
)
def _out_kernel(p0_ref, p1_ref, h2_ref, deg_ref, b_ref, o_ref):
    deg = 1.0 + jnp.sum(deg_ref[...], axis=0)
    dinv = lax.rsqrt(deg)
    o_ref[...] = (p0_ref[0] + p1_ref[0] + h2_ref[...]) * dinv[:, None] + b_ref[...]


def kernel(x, edge_index, edge_weight, W, b):
    assert x.shape == (N, F_IN) and edge_index.shape == (2, E)
    src = edge_index[0]
    dst = edge_index[1]
    x_pad = jnp.pad(x, ((0, N_PAD - N), (0, 0)))
    zeros = jnp.zeros((N_PAD, F_OUT), jnp.float32)

    deg_parts, sig = _deg_kernel(edge_weight, dst)
    h2 = _h2_kernel(x_pad, W, deg_parts)
    parts = _agg_kernel(h2, src, dst, sig, zeros)
    out_pad = _out_kernel(parts, parts, h2, deg_parts, b.reshape(1, F_OUT))
    return out_pad[:N]


# speedup vs baseline: 26.8003x; 1.9825x over previous
import dataclasses
import functools

import jax
import jax.numpy as jnp
from jax import lax
from jax.experimental import pallas as pl
from jax.experimental.pallas import tpu as pltpu
from jax.experimental.pallas import tpu_sc as plsc

N = 10000
E = 320000
F_IN = 128
F_OUT = 128

NC = 2
NS = 16
NW = NC * NS
L = 16

N_PAD = 10240
E_PER_W = E // NW
KA = 2000
KC = 80
ROWS_PER_TILE = N_PAD // NS
BLK = 256

_mesh = plsc.VectorSubcoreMesh(core_axis_name="c", subcore_axis_name="s")

_sc_params = pltpu.CompilerParams()
if "needs_layout_passes" in pltpu.CompilerParams.__dataclass_fields__:
    _sc_params = dataclasses.replace(_sc_params, needs_layout_passes=False)


@functools.partial(
    pl.kernel,
    out_type=(
        jax.ShapeDtypeStruct((NW, N_PAD), jnp.float32),
        jax.ShapeDtypeStruct((E,), jnp.float32),
    ),
    mesh=_mesh,
    scratch_types=[
        pltpu.VMEM((N_PAD,), jnp.float32),
        pltpu.VMEM((KA,), jnp.float32),
        pltpu.VMEM((KA,), jnp.int32),
        pltpu.VMEM((KA,), jnp.float32),
        pltpu.SemaphoreType.DMA,
    ],
    compiler_params=_sc_params,
)
def _deg_kernel(ew_hbm, dst_hbm, deg_out, sig_out, deg_v, ew_v, dst_v, sig_v, sem):
    w = lax.axis_index("c") * NS + lax.axis_index("s")
    base = w * E_PER_W
    zero16 = jnp.zeros((L,), jnp.float32)

    @pl.loop(0, N_PAD, step=L)
    def _(i):
        deg_v[pl.ds(i, L)] = zero16

    @pl.loop(0, E_PER_W, step=KA)
    def _(off):
        pltpu.async_copy(ew_hbm.at[pl.ds(base + off, KA)], ew_v, sem).wait()
        pltpu.async_copy(dst_hbm.at[pl.ds(base + off, KA)], dst_v, sem).wait()

        @pl.loop(0, KA, step=L)
        def _(j):
            wv = ew_v[pl.ds(j, L)]
            s = 1.0 / (1.0 + jnp.exp(-wv))
            sig_v[pl.ds(j, L)] = s
            di = dst_v[pl.ds(j, L)]
            plsc.addupdate_scatter(deg_v, [di], s)

        pltpu.async_copy(sig_v, sig_out.at[pl.ds(base + off, KA)], sem).wait()

    pltpu.async_copy(deg_v, deg_out.at[w], sem).wait()


@functools.partial(
    pl.pallas_call,
    grid=(N_PAD // BLK,),
    in_specs=[
        pl.BlockSpec((BLK, F_IN), lambda i: (i, 0)),
        pl.BlockSpec((F_IN, F_OUT), lambda i: (0, 0)),
        pl.BlockSpec((NW, BLK), lambda i: (0, i)),
    ],
    out_specs=pl.BlockSpec((BLK, F_OUT), lambda i: (i, 0)),
    out_shape=jax.ShapeDtypeStruct((N_PAD, F_OUT), jnp.float32),
)
def _h2_kernel(x_ref, w_ref, deg_ref, h2_ref):
    deg = 1.0 + jnp.sum(deg_ref[...], axis=0)
    dinv = lax.rsqrt(deg)
    h = jnp.dot(x_ref[...], w_ref[...], preferred_element_type=jnp.float32,
                precision=lax.Precision.HIGHEST)
    h2_ref[...] = h * dinv[:, None]


NBLK = E_PER_W // KC


@functools.partial(
    pl.kernel,
    out_type=jax.ShapeDtypeStruct((NC, N_PAD, F_OUT), jnp.float32),
    mesh=_mesh,
    scratch_types=[
        pltpu.VMEM_SHARED((N_PAD, F_OUT), jnp.float32),
        pltpu.VMEM((KC, F_OUT), jnp.float32),
        pltpu.VMEM((KC, F_OUT), jnp.float32),
        pltpu.VMEM((KC,), jnp.int32),
        pltpu.VMEM((KC,), jnp.int32),
        pltpu.VMEM((KC,), jnp.int32),
        pltpu.VMEM((KC,), jnp.int32),
        pltpu.VMEM((KC,), jnp.float32),
        pltpu.VMEM((KC,), jnp.float32),
        pltpu.SemaphoreType.DMA,
        pltpu.SemaphoreType.DMA,
        pltpu.SemaphoreType.DMA,
        pltpu.SemaphoreType.DMA,
        pltpu.SemaphoreType.DMA,
        pltpu.SemaphoreType.DMA,
        pltpu.SemaphoreType.DMA,
        pltpu.SemaphoreType.DMA,
    ],
    compiler_params=_sc_params,
)
def _agg_kernel(h2_hbm, src_hbm, dst_hbm, sig_hbm, out_hbm,
                acc_sh, rows0, rows1, src0, src1, dst0, dst1, sig0, sig1,
                sem_i0, sem_i1, sem_d0, sem_d1,
                sem_g0, sem_g1, sem_s0, sem_s1):
    c = lax.axis_index("c")
    s = lax.axis_index("s")
    w = c * NS + s
    base = w * E_PER_W
    row0_ = s * ROWS_PER_TILE

    rows = (rows0, rows1)
    src = (src0, src1)
    dst = (dst0, dst1)
    sig = (sig0, sig1)
    sem_i = (sem_i0, sem_i1)
    sem_d = (sem_d0, sem_d1)
    sem_g = (sem_g0, sem_g1)
    sem_s = (sem_s0, sem_s1)

    def issue_idx(b_off, sl):
        pltpu.async_copy(src_hbm.at[pl.ds(base + b_off, KC)], src[sl], sem_i[sl])
        pltpu.async_copy(sig_hbm.at[pl.ds(base + b_off, KC)], sig[sl], sem_i[sl])

    def wait_idx(sl):
        pltpu.make_async_copy(src_hbm.at[pl.ds(0, KC)], src[sl], sem_i[sl]).wait()
        pltpu.make_async_copy(sig_hbm.at[pl.ds(0, KC)], sig[sl], sem_i[sl]).wait()

    def issue_dst(b_off, sl):
        pltpu.async_copy(dst_hbm.at[pl.ds(base + b_off, KC)], dst[sl], sem_d[sl])

    def wait_dst(sl):
        pltpu.make_async_copy(dst_hbm.at[pl.ds(0, KC)], dst[sl], sem_d[sl]).wait()

    def issue_gather(sl):
        pltpu.async_copy(h2_hbm.at[src[sl]], rows[sl], sem_g[sl])

    def wait_gather(sl):
        pltpu.make_async_copy(h2_hbm.at[src[sl]], rows[sl], sem_g[sl]).wait()

    def issue_scatter(sl):
        pltpu.async_copy(rows[sl], acc_sh.at[dst[sl]], sem_s[sl], add=True)

    def wait_scatter(sl):
        pltpu.make_async_copy(rows[sl], acc_sh.at[dst[sl]], sem_s[sl]).wait()

    def scale(sl):
        rv = rows[sl]
        sv_ref = sig[sl]

        @pl.loop(0, KC, unroll=2)
        def _(k):
            kv = jnp.broadcast_to(k, (L,)).astype(jnp.int32)
            sv = plsc.load_gather(sv_ref, [kv])
            for ccol in range(F_OUT // L):
                slc = pl.ds(ccol * L, L)
                rv[k, slc] = rv[k, slc] * sv

    issue_idx(0, 0)
    issue_idx(KC, 1)

    z16 = jnp.zeros((L,), jnp.float32)

    @pl.loop(0, KC)
    def _(k):
        for ccol in range(F_OUT // L):
            rows0[k, pl.ds(ccol * L, L)] = z16

    for t in range(ROWS_PER_TILE // KC):
        pltpu.sync_copy(rows0, acc_sh.at[pl.ds(row0_ + t * KC, KC)])
    plsc.subcore_barrier()

    issue_dst(0, 0)
    wait_idx(0)
    issue_gather(0)
    issue_dst(KC, 1)
    wait_idx(1)
    issue_gather(1)
    wait_gather(0)
    scale(0)
    issue_idx(2 * KC, 0)
    wait_dst(0)
    issue_scatter(0)
    wait_gather(1)
    scale(1)
    issue_idx(3 * KC, 1)
    wait_dst(1)
    issue_scatter(1)

    @pl.loop(1, (NBLK - 1) // 2)
    def _(i):
        b0 = 2 * i * KC
        b1 = b0 + KC
        wait_scatter(0)
        issue_dst(b0, 0)
        wait_idx(0)
        issue_gather(0)
        wait_scatter(1)
        issue_dst(b1, 1)
        wait_idx(1)
        issue_gather(1)
        wait_gather(0)
        scale(0)
        issue_idx(b0 + 2 * KC, 0)
        wait_dst(0)
        issue_scatter(0)
        wait_gather(1)
        scale(1)

        @pl.when(b1 + 2 * KC < E_PER_W)
        def _():
            issue_idx(b1 + 2 * KC, 1)

        wait_dst(1)
        issue_scatter(1)

    b_last = (NBLK - 1) * KC
    wait_scatter(0)
    issue_dst(b_last, 0)
    wait_idx(0)
    issue_gather(0)
    wait_gather(0)
    scale(0)
    wait_dst(0)
    issue_scatter(0)
    wait_scatter(0)
    wait_scatter(1)

    plsc.subcore_barrier()
    pltpu.async_copy(acc_sh.at[pl.ds(row0_, ROWS_PER_TILE)],
                     out_hbm.at[c, pl.ds(row0_, ROWS_PER_TILE)], sem_g0).wait()


@functools.partial(
    pl.pallas_call,
    grid=(N_PAD // BLK,),
    in_specs=[
        pl.BlockSpec((1, BLK, F_OUT), lambda i: (0, i, 0)),
        pl.BlockSpec((1, BLK, F_OUT), lambda i: (1, i, 0)),
        pl.BlockSpec((BLK, F_OUT), lambda i: (i, 0)),
        pl.BlockSpec((NW, BLK), lambda i: (0, i)),
        pl.BlockSpec((1, F_OUT), lambda i: (0, 0)),
    ],
    out_specs=pl.BlockSpec((BLK, F_OUT), lambda i: (i, 0)),
    out_shape=jax.ShapeDtypeStruct((N_PAD, F_OUT), jnp.float32),
)
def _out_kernel(p0_ref, p1_ref, h2_ref, deg_ref, b_ref, o_ref):
    deg = 1.0 + jnp.sum(deg_ref[...], axis=0)
    dinv = lax.rsqrt(deg)
    o_ref[...] = (p0_ref[0] + p1_ref[0] + h2_ref[...]) * dinv[:, None] + b_ref[...]


def kernel(x, edge_index, edge_weight, W, b):
    assert x.shape == (N, F_IN) and edge_index.shape == (2, E)
    src = edge_index[0]
    dst = edge_index[1]
    x_pad = jnp.pad(x, ((0, N_PAD - N), (0, 0)))

    deg_parts, sig = _deg_kernel(edge_weight, dst)
    h2 = _h2_kernel(x_pad, W, deg_parts)
    parts = _agg_kernel(h2, src, dst, sig)
    out_pad = _out_kernel(parts, parts, h2, deg_parts, b.reshape(1, F_OUT))
    return out_pad[:N]


# speedup vs baseline: 27.4761x; 1.0252x over previous
import dataclasses
import functools

import jax
import jax.numpy as jnp
from jax import lax
from jax.experimental import pallas as pl
from jax.experimental.pallas import tpu as pltpu
from jax.experimental.pallas import tpu_sc as plsc

N = 10000
E = 320000
F_IN = 128
F_OUT = 128

NC = 2
NS = 16
NW = NC * NS
L = 16

N_PAD = 10240
E_PER_W = E // NW
KA = 2000
KC = 80
ROWS_PER_TILE = N_PAD // NS
BLK = 256

_mesh = plsc.VectorSubcoreMesh(core_axis_name="c", subcore_axis_name="s")

_sc_params = pltpu.CompilerParams()
if "needs_layout_passes" in pltpu.CompilerParams.__dataclass_fields__:
    _sc_params = dataclasses.replace(_sc_params, needs_layout_passes=False)


@functools.partial(
    pl.kernel,
    out_type=(
        jax.ShapeDtypeStruct((NW, N_PAD), jnp.float32),
        jax.ShapeDtypeStruct((E,), jnp.float32),
    ),
    mesh=_mesh,
    scratch_types=[
        pltpu.VMEM((N_PAD,), jnp.float32),
        pltpu.VMEM((KA,), jnp.float32),
        pltpu.VMEM((KA,), jnp.float32),
        pltpu.VMEM((KA,), jnp.int32),
        pltpu.VMEM((KA,), jnp.int32),
        pltpu.VMEM((KA,), jnp.float32),
        pltpu.VMEM((KA,), jnp.float32),
        pltpu.SemaphoreType.DMA,
        pltpu.SemaphoreType.DMA,
        pltpu.SemaphoreType.DMA,
        pltpu.SemaphoreType.DMA,
        pltpu.SemaphoreType.DMA,
    ],
    compiler_params=_sc_params,
)
def _deg_kernel(ew_hbm, dst_hbm, deg_out, sig_out, deg_v,
                ew0, ew1, dst0, dst1, sig0, sig1,
                sem_a0, sem_a1, sem_w0, sem_w1, sem_f):
    w = lax.axis_index("c") * NS + lax.axis_index("s")
    base = w * E_PER_W
    zero16 = jnp.zeros((L,), jnp.float32)

    ew = (ew0, ew1)
    dst = (dst0, dst1)
    sig = (sig0, sig1)
    sem_a = (sem_a0, sem_a1)
    sem_w = (sem_w0, sem_w1)
    nblk = E_PER_W // KA

    def issue_in(i):
        sl = i % 2
        off = base + i * KA
        pltpu.async_copy(ew_hbm.at[pl.ds(off, KA)], ew[sl], sem_a[sl])
        pltpu.async_copy(dst_hbm.at[pl.ds(off, KA)], dst[sl], sem_a[sl])

    issue_in(0)
    issue_in(1)

    @pl.loop(0, N_PAD, step=L, unroll=4)
    def _(i):
        deg_v[pl.ds(i, L)] = zero16

    for i in range(nblk):
        sl = i % 2
        pltpu.make_async_copy(ew_hbm.at[pl.ds(0, KA)], ew[sl], sem_a[sl]).wait()
        pltpu.make_async_copy(dst_hbm.at[pl.ds(0, KA)], dst[sl], sem_a[sl]).wait()
        if i >= 2:
            pltpu.make_async_copy(sig[sl], sig_out.at[pl.ds(0, KA)],
                                  sem_w[sl]).wait()

        @pl.loop(0, KA, step=L, unroll=2)
        def _(j):
            wv = ew[sl][pl.ds(j, L)]
            s = 1.0 / (1.0 + jnp.exp(-wv))
            sig[sl][pl.ds(j, L)] = s
            di = dst[sl][pl.ds(j, L)]
            plsc.addupdate_scatter(deg_v, [di], s)

        pltpu.async_copy(sig[sl], sig_out.at[pl.ds(base + i * KA, KA)], sem_w[sl])
        if i + 2 < nblk:
            issue_in(i + 2)

    pltpu.make_async_copy(sig0, sig_out.at[pl.ds(0, KA)], sem_w[nblk % 2]).wait()
    pltpu.make_async_copy(sig1, sig_out.at[pl.ds(0, KA)], sem_w[(nblk + 1) % 2]).wait()
    pltpu.async_copy(deg_v, deg_out.at[w], sem_f).wait()


@functools.partial(
    pl.pallas_call,
    grid=(N_PAD // BLK,),
    in_specs=[
        pl.BlockSpec((BLK, F_IN), lambda i: (i, 0)),
        pl.BlockSpec((F_IN, F_OUT), lambda i: (0, 0)),
        pl.BlockSpec((NW, BLK), lambda i: (0, i)),
    ],
    out_specs=pl.BlockSpec((BLK, F_OUT), lambda i: (i, 0)),
    out_shape=jax.ShapeDtypeStruct((N_PAD, F_OUT), jnp.float32),
)
def _h2_kernel(x_ref, w_ref, deg_ref, h2_ref):
    deg = 1.0 + jnp.sum(deg_ref[...], axis=0)
    dinv = lax.rsqrt(deg)
    h = jnp.dot(x_ref[...], w_ref[...], preferred_element_type=jnp.float32,
                precision=lax.Precision.HIGHEST)
    h2_ref[...] = h * dinv[:, None]


NBLK = E_PER_W // KC


@functools.partial(
    pl.kernel,
    out_type=jax.ShapeDtypeStruct((NC, N_PAD, F_OUT), jnp.float32),
    mesh=_mesh,
    scratch_types=[
        pltpu.VMEM_SHARED((N_PAD, F_OUT), jnp.float32),
        pltpu.VMEM((KC, F_OUT), jnp.float32),
        pltpu.VMEM((KC, F_OUT), jnp.float32),
        pltpu.VMEM((KC,), jnp.int32),
        pltpu.VMEM((KC,), jnp.int32),
        pltpu.VMEM((KC,), jnp.int32),
        pltpu.VMEM((KC,), jnp.int32),
        pltpu.VMEM((KC,), jnp.float32),
        pltpu.VMEM((KC,), jnp.float32),
        pltpu.SemaphoreType.DMA,
        pltpu.SemaphoreType.DMA,
        pltpu.SemaphoreType.DMA,
        pltpu.SemaphoreType.DMA,
        pltpu.SemaphoreType.DMA,
        pltpu.SemaphoreType.DMA,
        pltpu.SemaphoreType.DMA,
        pltpu.SemaphoreType.DMA,
    ],
    compiler_params=_sc_params,
)
def _agg_kernel(h2_hbm, src_hbm, dst_hbm, sig_hbm, out_hbm,
                acc_sh, rows0, rows1, src0, src1, dst0, dst1, sig0, sig1,
                sem_i0, sem_i1, sem_d0, sem_d1,
                sem_g0, sem_g1, sem_s0, sem_s1):
    c = lax.axis_index("c")
    s = lax.axis_index("s")
    w = c * NS + s
    base = w * E_PER_W
    row0_ = s * ROWS_PER_TILE

    rows = (rows0, rows1)
    src = (src0, src1)
    dst = (dst0, dst1)
    sig = (sig0, sig1)
    sem_i = (sem_i0, sem_i1)
    sem_d = (sem_d0, sem_d1)
    sem_g = (sem_g0, sem_g1)
    sem_s = (sem_s0, sem_s1)

    def issue_idx(b_off, sl):
        pltpu.async_copy(src_hbm.at[pl.ds(base + b_off, KC)], src[sl], sem_i[sl])
        pltpu.async_copy(sig_hbm.at[pl.ds(base + b_off, KC)], sig[sl], sem_i[sl])

    def wait_idx(sl):
        pltpu.make_async_copy(src_hbm.at[pl.ds(0, KC)], src[sl], sem_i[sl]).wait()
        pltpu.make_async_copy(sig_hbm.at[pl.ds(0, KC)], sig[sl], sem_i[sl]).wait()

    def issue_dst(b_off, sl):
        pltpu.async_copy(dst_hbm.at[pl.ds(base + b_off, KC)], dst[sl], sem_d[sl])

    def wait_dst(sl):
        pltpu.make_async_copy(dst_hbm.at[pl.ds(0, KC)], dst[sl], sem_d[sl]).wait()

    def issue_gather(sl):
        pltpu.async_copy(h2_hbm.at[src[sl]], rows[sl], sem_g[sl])

    def wait_gather(sl):
        pltpu.make_async_copy(h2_hbm.at[src[sl]], rows[sl], sem_g[sl]).wait()

    def issue_scatter(sl):
        pltpu.async_copy(rows[sl], acc_sh.at[dst[sl]], sem_s[sl], add=True)

    def wait_scatter(sl):
        pltpu.make_async_copy(rows[sl], acc_sh.at[dst[sl]], sem_s[sl]).wait()

    def scale(sl):
        rv = rows[sl]
        sv_ref = sig[sl]

        @pl.loop(0, KC, unroll=4)
        def _(k):
            kv = jnp.broadcast_to(k, (L,)).astype(jnp.int32)
            sv = plsc.load_gather(sv_ref, [kv])
            for ccol in range(F_OUT // L):
                slc = pl.ds(ccol * L, L)
                rv[k, slc] = rv[k, slc] * sv

    issue_idx(0, 0)
    issue_idx(KC, 1)

    z16 = jnp.zeros((L,), jnp.float32)

    @pl.loop(0, KC)
    def _(k):
        for ccol in range(F_OUT // L):
            rows0[k, pl.ds(ccol * L, L)] = z16

    for t in range(ROWS_PER_TILE // KC):
        pltpu.sync_copy(rows0, acc_sh.at[pl.ds(row0_ + t * KC, KC)])
    plsc.subcore_barrier()

    issue_dst(0, 0)
    wait_idx(0)
    issue_gather(0)
    issue_dst(KC, 1)
    wait_idx(1)
    issue_gather(1)
    wait_gather(0)
    scale(0)
    issue_idx(2 * KC, 0)
    wait_dst(0)
    issue_scatter(0)
    wait_gather(1)
    scale(1)
    issue_idx(3 * KC, 1)
    wait_dst(1)
    issue_scatter(1)

    @pl.loop(1, (NBLK - 1) // 2)
    def _(i):
        b0 = 2 * i * KC
        b1 = b0 + KC
        wait_scatter(0)
        issue_dst(b0, 0)
        wait_idx(0)
        issue_gather(0)
        wait_scatter(1)
        issue_dst(b1, 1)
        wait_idx(1)
        issue_gather(1)
        wait_gather(0)
        scale(0)
        issue_idx(b0 + 2 * KC, 0)
        wait_dst(0)
        issue_scatter(0)
        wait_gather(1)
        scale(1)

        @pl.when(b1 + 2 * KC < E_PER_W)
        def _():
            issue_idx(b1 + 2 * KC, 1)

        wait_dst(1)
        issue_scatter(1)

    b_last = (NBLK - 1) * KC
    wait_scatter(0)
    issue_dst(b_last, 0)
    wait_idx(0)
    issue_gather(0)
    wait_gather(0)
    scale(0)
    wait_dst(0)
    issue_scatter(0)
    wait_scatter(0)
    wait_scatter(1)

    plsc.subcore_barrier()
    pltpu.async_copy(acc_sh.at[pl.ds(row0_, ROWS_PER_TILE)],
                     out_hbm.at[c, pl.ds(row0_, ROWS_PER_TILE)], sem_g0).wait()


@functools.partial(
    pl.pallas_call,
    grid=(N_PAD // BLK,),
    in_specs=[
        pl.BlockSpec((1, BLK, F_OUT), lambda i: (0, i, 0)),
        pl.BlockSpec((1, BLK, F_OUT), lambda i: (1, i, 0)),
        pl.BlockSpec((BLK, F_OUT), lambda i: (i, 0)),
        pl.BlockSpec((NW, BLK), lambda i: (0, i)),
        pl.BlockSpec((1, F_OUT), lambda i: (0, 0)),
    ],
    out_specs=pl.BlockSpec((BLK, F_OUT), lambda i: (i, 0)),
    out_shape=jax.ShapeDtypeStruct((N_PAD, F_OUT), jnp.float32),
)
def _out_kernel(p0_ref, p1_ref, h2_ref, deg_ref, b_ref, o_ref):
    deg = 1.0 + jnp.sum(deg_ref[...], axis=0)
    dinv = lax.rsqrt(deg)
    o_ref[...] = (p0_ref[0] + p1_ref[0] + h2_ref[...]) * dinv[:, None] + b_ref[...]


def kernel(x, edge_index, edge_weight, W, b):
    assert x.shape == (N, F_IN) and edge_index.shape == (2, E)
    src = edge_index[0]
    dst = edge_index[1]
    x_pad = jnp.pad(x, ((0, N_PAD - N), (0, 0)))

    deg_parts, sig = _deg_kernel(edge_weight, dst)
    h2 = _h2_kernel(x_pad, W, deg_parts)
    parts = _agg_kernel(h2, src, dst, sig)
    out_pad = _out_kernel(parts, parts, h2, deg_parts, b.reshape(1, F_OUT))
    return out_pad[:N]


# speedup vs baseline: 31.5189x; 1.1471x over previous
import dataclasses
import functools

import jax
import jax.numpy as jnp
from jax import lax
from jax.experimental import pallas as pl
from jax.experimental.pallas import tpu as pltpu
from jax.experimental.pallas import tpu_sc as plsc

N = 10000
E = 320000
F_IN = 128
F_OUT = 128

NC = 2
NS = 16
NW = NC * NS
L = 16

N_PAD = 10240
E_PER_W = E // NW
KA = 2000
KC = 400
ROWS_PER_TILE = N_PAD // NS
BLK = 256

_mesh = plsc.VectorSubcoreMesh(core_axis_name="c", subcore_axis_name="s")

_sc_params = pltpu.CompilerParams()
if "needs_layout_passes" in pltpu.CompilerParams.__dataclass_fields__:
    _sc_params = dataclasses.replace(_sc_params, needs_layout_passes=False)


@functools.partial(
    pl.kernel,
    out_type=(
        jax.ShapeDtypeStruct((NW, N_PAD), jnp.float32),
        jax.ShapeDtypeStruct((E,), jnp.float32),
    ),
    mesh=_mesh,
    scratch_types=[
        pltpu.VMEM((N_PAD,), jnp.float32),
        pltpu.VMEM((KA,), jnp.float32),
        pltpu.VMEM((KA,), jnp.float32),
        pltpu.VMEM((KA,), jnp.int32),
        pltpu.VMEM((KA,), jnp.int32),
        pltpu.VMEM((KA,), jnp.float32),
        pltpu.VMEM((KA,), jnp.float32),
        pltpu.SemaphoreType.DMA,
        pltpu.SemaphoreType.DMA,
        pltpu.SemaphoreType.DMA,
        pltpu.SemaphoreType.DMA,
        pltpu.SemaphoreType.DMA,
    ],
    compiler_params=_sc_params,
)
def _deg_kernel(ew_hbm, dst_hbm, deg_out, sig_out, deg_v,
                ew0, ew1, dst0, dst1, sig0, sig1,
                sem_a0, sem_a1, sem_w0, sem_w1, sem_f):
    w = lax.axis_index("c") * NS + lax.axis_index("s")
    base = w * E_PER_W
    zero16 = jnp.zeros((L,), jnp.float32)

    ew = (ew0, ew1)
    dst = (dst0, dst1)
    sig = (sig0, sig1)
    sem_a = (sem_a0, sem_a1)
    sem_w = (sem_w0, sem_w1)
    nblk = E_PER_W // KA

    def issue_in(i):
        sl = i % 2
        off = base + i * KA
        pltpu.async_copy(ew_hbm.at[pl.ds(off, KA)], ew[sl], sem_a[sl])
        pltpu.async_copy(dst_hbm.at[pl.ds(off, KA)], dst[sl], sem_a[sl])

    issue_in(0)
    issue_in(1)

    @pl.loop(0, N_PAD, step=L, unroll=4)
    def _(i):
        deg_v[pl.ds(i, L)] = zero16

    for i in range(nblk):
        sl = i % 2
        pltpu.make_async_copy(ew_hbm.at[pl.ds(0, KA)], ew[sl], sem_a[sl]).wait()
        pltpu.make_async_copy(dst_hbm.at[pl.ds(0, KA)], dst[sl], sem_a[sl]).wait()
        if i >= 2:
            pltpu.make_async_copy(sig[sl], sig_out.at[pl.ds(0, KA)],
                                  sem_w[sl]).wait()

        @pl.loop(0, KA, step=L, unroll=2)
        def _(j):
            wv = ew[sl][pl.ds(j, L)]
            s = 1.0 / (1.0 + jnp.exp(-wv))
            sig[sl][pl.ds(j, L)] = s
            di = dst[sl][pl.ds(j, L)]
            plsc.addupdate_scatter(deg_v, [di], s)

        pltpu.async_copy(sig[sl], sig_out.at[pl.ds(base + i * KA, KA)], sem_w[sl])
        if i + 2 < nblk:
            issue_in(i + 2)

    pltpu.make_async_copy(sig0, sig_out.at[pl.ds(0, KA)], sem_w[nblk % 2]).wait()
    pltpu.make_async_copy(sig1, sig_out.at[pl.ds(0, KA)], sem_w[(nblk + 1) % 2]).wait()
    pltpu.async_copy(deg_v, deg_out.at[w], sem_f).wait()


@functools.partial(
    pl.pallas_call,
    grid=(N_PAD // BLK,),
    in_specs=[
        pl.BlockSpec((BLK, F_IN), lambda i: (i, 0)),
        pl.BlockSpec((F_IN, F_OUT), lambda i: (0, 0)),
        pl.BlockSpec((NW, BLK), lambda i: (0, i)),
    ],
    out_specs=pl.BlockSpec((BLK, F_OUT), lambda i: (i, 0)),
    out_shape=jax.ShapeDtypeStruct((N_PAD, F_OUT), jnp.float32),
)
def _h2_kernel(x_ref, w_ref, deg_ref, h2_ref):
    deg = 1.0 + jnp.sum(deg_ref[...], axis=0)
    dinv = lax.rsqrt(deg)
    h = jnp.dot(x_ref[...], w_ref[...], preferred_element_type=jnp.float32,
                precision=lax.Precision.HIGHEST)
    h2_ref[...] = h * dinv[:, None]


SW = 80
CPB = KC // SW
NBLK = E_PER_W // KC
NCHUNK = E_PER_W // SW
RSLOTS = 4


@functools.partial(
    pl.kernel,
    out_type=jax.ShapeDtypeStruct((NC, N_PAD, F_OUT), jnp.float32),
    mesh=_mesh,
    scratch_types=[
        pltpu.VMEM_SHARED((N_PAD, F_OUT), jnp.float32),
        pltpu.VMEM((SW, F_OUT), jnp.float32),
        pltpu.VMEM((SW, F_OUT), jnp.float32),
        pltpu.VMEM((SW, F_OUT), jnp.float32),
        pltpu.VMEM((SW, F_OUT), jnp.float32),
        pltpu.VMEM((KC,), jnp.int32),
        pltpu.VMEM((KC,), jnp.int32),
        pltpu.VMEM((KC,), jnp.int32),
        pltpu.VMEM((KC,), jnp.int32),
        pltpu.VMEM((KC,), jnp.float32),
        pltpu.VMEM((KC,), jnp.float32),
        pltpu.SemaphoreType.DMA,
        pltpu.SemaphoreType.DMA,
        pltpu.SemaphoreType.DMA,
        pltpu.SemaphoreType.DMA,
        pltpu.SemaphoreType.DMA,
        pltpu.SemaphoreType.DMA,
        pltpu.SemaphoreType.DMA,
        pltpu.SemaphoreType.DMA,
        pltpu.SemaphoreType.DMA,
        pltpu.SemaphoreType.DMA,
    ],
    compiler_params=_sc_params,
)
def _agg_kernel(h2_hbm, src_hbm, dst_hbm, sig_hbm, out_hbm,
                acc_sh, rows0, rows1, rows2, rows3,
                src0, src1, dst0, dst1, sig0, sig1,
                sem_i0, sem_i1,
                sem_g0, sem_g1, sem_g2, sem_g3,
                sem_s0, sem_s1, sem_s2, sem_s3):
    c = lax.axis_index("c")
    s = lax.axis_index("s")
    w = c * NS + s
    base = w * E_PER_W
    row0_ = s * ROWS_PER_TILE

    rows = (rows0, rows1, rows2, rows3)
    src = (src0, src1)
    dst = (dst0, dst1)
    sig = (sig0, sig1)
    sem_i = (sem_i0, sem_i1)
    sem_g = (sem_g0, sem_g1, sem_g2, sem_g3)
    sem_s = (sem_s0, sem_s1, sem_s2, sem_s3)

    def issue_idx(b, isl):
        off = base + b * KC
        pltpu.async_copy(src_hbm.at[pl.ds(off, KC)], src[isl], sem_i[isl])
        pltpu.async_copy(dst_hbm.at[pl.ds(off, KC)], dst[isl], sem_i[isl])
        pltpu.async_copy(sig_hbm.at[pl.ds(off, KC)], sig[isl], sem_i[isl])

    def wait_idx(isl):
        pltpu.make_async_copy(src_hbm.at[pl.ds(0, KC)], src[isl],
                              sem_i[isl]).wait()
        pltpu.make_async_copy(dst_hbm.at[pl.ds(0, KC)], dst[isl],
                              sem_i[isl]).wait()
        pltpu.make_async_copy(sig_hbm.at[pl.ds(0, KC)], sig[isl],
                              sem_i[isl]).wait()

    def issue_gather(rsl, isl, q):
        pltpu.async_copy(h2_hbm.at[src[isl].at[pl.ds(SW * q, SW)]],
                         rows[rsl], sem_g[rsl])

    def wait_gather(rsl, isl, q):
        pltpu.make_async_copy(h2_hbm.at[src[isl].at[pl.ds(SW * q, SW)]],
                              rows[rsl], sem_g[rsl]).wait()

    def issue_scatter(rsl, isl, q):
        pltpu.async_copy(rows[rsl], acc_sh.at[dst[isl].at[pl.ds(SW * q, SW)]],
                         sem_s[rsl], add=True)

    def wait_scatter(rsl, isl, q):
        pltpu.make_async_copy(rows[rsl], acc_sh.at[dst[isl].at[pl.ds(SW * q, SW)]],
                              sem_s[rsl]).wait()

    def scale(rsl, isl, q):
        rv = rows[rsl]
        sv_ref = sig[isl]

        @pl.loop(0, SW, unroll=4)
        def _(k):
            kv = jnp.broadcast_to(SW * q + k, (L,)).astype(jnp.int32)
            sv = plsc.load_gather(sv_ref, [kv])
            for ccol in range(F_OUT // L):
                slc = pl.ds(ccol * L, L)
                rv[k, slc] = rv[k, slc] * sv

    def chunk(q, rsl, isl, q1, rsl1, isl1, *, idx_wait=False, drain=True,
              gissue=True, gissue_pred=None, idx_issue_b=None, idx_issue_sl=0):
        if drain:
            wait_scatter(rsl1, isl1, q1)
        if gissue:
            def _prefetch():
                if idx_wait:
                    wait_idx(isl1)
                issue_gather(rsl1, isl1, q1)
            if gissue_pred is None:
                _prefetch()
            else:
                pl.when(gissue_pred)(_prefetch)
        wait_gather(rsl, isl, q)
        scale(rsl, isl, q)
        issue_scatter(rsl, isl, q)
        if idx_issue_b is not None:
            @pl.when(idx_issue_b < NBLK)
            def _():
                issue_idx(idx_issue_b, idx_issue_sl)

    issue_idx(0, 0)
    issue_idx(1, 1)

    z16 = jnp.zeros((L,), jnp.float32)

    @pl.loop(0, SW, unroll=2)
    def _(k):
        for ccol in range(F_OUT // L):
            rows0[k, pl.ds(ccol * L, L)] = z16

    for tt in range(ROWS_PER_TILE // SW):
        pltpu.sync_copy(rows0, acc_sh.at[pl.ds(row0_ + tt * SW, SW)])
    plsc.subcore_barrier()

    wait_idx(0)
    issue_gather(0, 0, 0)

    chunk(0, 0, 0, 1, 1, 0, drain=False)
    chunk(1, 1, 0, 2, 2, 0, drain=False)
    chunk(2, 2, 0, 3, 3, 0, drain=False)
    chunk(3, 3, 0, 4, 0, 0)
    chunk(4, 0, 0, 0, 1, 1, idx_wait=True)

    NG = (NBLK - 1) // 4

    @pl.loop(0, NG)
    def _(g):
        for j in range(20):
            q = j % 5
            blk_rel = 1 + j // 5
            isl = blk_rel % 2
            rsl = (5 + j) % 4
            q1 = (j + 1) % 5
            isl1 = (1 + (j + 1) // 5) % 2
            rsl1 = (6 + j) % 4
            b_tr = 4 * g + blk_rel
            kw = {}
            if q == 4:
                kw["idx_wait"] = True
            if j == 19:
                kw["gissue_pred"] = g < NG - 1
            if q == 2:
                kw["idx_issue_b"] = b_tr + 1
                kw["idx_issue_sl"] = (blk_rel + 1) % 2
            chunk(q, rsl, isl, q1, rsl1, isl1, **kw)

    wait_scatter(2, 0, 2)
    wait_scatter(3, 0, 3)
    wait_scatter(0, 0, 4)

    plsc.subcore_barrier()
    pltpu.async_copy(acc_sh.at[pl.ds(row0_, ROWS_PER_TILE)],
                     out_hbm.at[c, pl.ds(row0_, ROWS_PER_TILE)], sem_g0).wait()


@functools.partial(
    pl.pallas_call,
    grid=(N_PAD // BLK,),
    in_specs=[
        pl.BlockSpec((1, BLK, F_OUT), lambda i: (0, i, 0)),
        pl.BlockSpec((1, BLK, F_OUT), lambda i: (1, i, 0)),
        pl.BlockSpec((BLK, F_OUT), lambda i: (i, 0)),
        pl.BlockSpec((NW, BLK), lambda i: (0, i)),
        pl.BlockSpec((1, F_OUT), lambda i: (0, 0)),
    ],
    out_specs=pl.BlockSpec((BLK, F_OUT), lambda i: (i, 0)),
    out_shape=jax.ShapeDtypeStruct((N_PAD, F_OUT), jnp.float32),
)
def _out_kernel(p0_ref, p1_ref, h2_ref, deg_ref, b_ref, o_ref):
    deg = 1.0 + jnp.sum(deg_ref[...], axis=0)
    dinv = lax.rsqrt(deg)
    o_ref[...] = (p0_ref[0] + p1_ref[0] + h2_ref[...]) * dinv[:, None] + b_ref[...]


def kernel(x, edge_index, edge_weight, W, b):
    assert x.shape == (N, F_IN) and edge_index.shape == (2, E)
    src = edge_index[0]
    dst = edge_index[1]
    x_pad = jnp.pad(x, ((0, N_PAD - N), (0, 0)))

    deg_parts, sig = _deg_kernel(edge_weight, dst)
    h2 = _h2_kernel(x_pad, W, deg_parts)
    parts = _agg_kernel(h2, src, dst, sig)
    out_pad = _out_kernel(parts, parts, h2, deg_parts, b.reshape(1, F_OUT))
    return out_pad[:N]


# speedup vs baseline: 38.8518x; 1.2327x over previous
import dataclasses
import functools

import jax
import jax.numpy as jnp
from jax import lax
from jax.experimental import pallas as pl
from jax.experimental.pallas import tpu as pltpu
from jax.experimental.pallas import tpu_sc as plsc

N = 10000
E = 320000
F_IN = 128
F_OUT = 128

NC = 2
NS = 16
NW = NC * NS
L = 16

N_PAD = 10240
E_PER_W = E // NW
KA = 2000
KC = 400
ROWS_PER_TILE = N_PAD // NS
BLK = 2048

_mesh = plsc.VectorSubcoreMesh(core_axis_name="c", subcore_axis_name="s")

_sc_params = pltpu.CompilerParams()
if "needs_layout_passes" in pltpu.CompilerParams.__dataclass_fields__:
    _sc_params = dataclasses.replace(_sc_params, needs_layout_passes=False)


@functools.partial(
    pl.kernel,
    out_type=(
        jax.ShapeDtypeStruct((NW, N_PAD), jnp.float32),
        jax.ShapeDtypeStruct((E,), jnp.float32),
    ),
    mesh=_mesh,
    scratch_types=[
        pltpu.VMEM((N_PAD,), jnp.float32),
        pltpu.VMEM((KA,), jnp.float32),
        pltpu.VMEM((KA,), jnp.float32),
        pltpu.VMEM((KA,), jnp.int32),
        pltpu.VMEM((KA,), jnp.int32),
        pltpu.VMEM((KA,), jnp.float32),
        pltpu.VMEM((KA,), jnp.float32),
        pltpu.SemaphoreType.DMA,
        pltpu.SemaphoreType.DMA,
        pltpu.SemaphoreType.DMA,
        pltpu.SemaphoreType.DMA,
        pltpu.SemaphoreType.DMA,
    ],
    compiler_params=_sc_params,
)
def _deg_kernel(ew_hbm, dst_hbm, deg_out, sig_out, deg_v,
                ew0, ew1, dst0, dst1, sig0, sig1,
                sem_a0, sem_a1, sem_w0, sem_w1, sem_f):
    w = lax.axis_index("c") * NS + lax.axis_index("s")
    base = w * E_PER_W
    zero16 = jnp.zeros((L,), jnp.float32)

    ew = (ew0, ew1)
    dst = (dst0, dst1)
    sig = (sig0, sig1)
    sem_a = (sem_a0, sem_a1)
    sem_w = (sem_w0, sem_w1)
    nblk = E_PER_W // KA

    def issue_in(i):
        sl = i % 2
        off = base + i * KA
        pltpu.async_copy(ew_hbm.at[pl.ds(off, KA)], ew[sl], sem_a[sl])
        pltpu.async_copy(dst_hbm.at[pl.ds(off, KA)], dst[sl], sem_a[sl])

    issue_in(0)
    issue_in(1)

    @pl.loop(0, N_PAD, step=L, unroll=4)
    def _(i):
        deg_v[pl.ds(i, L)] = zero16

    for i in range(nblk):
        sl = i % 2
        pltpu.make_async_copy(ew_hbm.at[pl.ds(0, KA)], ew[sl], sem_a[sl]).wait()
        pltpu.make_async_copy(dst_hbm.at[pl.ds(0, KA)], dst[sl], sem_a[sl]).wait()
        if i >= 2:
            pltpu.make_async_copy(sig[sl], sig_out.at[pl.ds(0, KA)],
                                  sem_w[sl]).wait()

        @pl.loop(0, KA, step=L, unroll=2)
        def _(j):
            wv = ew[sl][pl.ds(j, L)]
            s = 1.0 / (1.0 + jnp.exp(-wv))
            sig[sl][pl.ds(j, L)] = s
            di = dst[sl][pl.ds(j, L)]
            plsc.addupdate_scatter(deg_v, [di], s)

        pltpu.async_copy(sig[sl], sig_out.at[pl.ds(base + i * KA, KA)], sem_w[sl])
        if i + 2 < nblk:
            issue_in(i + 2)

    pltpu.make_async_copy(sig0, sig_out.at[pl.ds(0, KA)], sem_w[nblk % 2]).wait()
    pltpu.make_async_copy(sig1, sig_out.at[pl.ds(0, KA)], sem_w[(nblk + 1) % 2]).wait()
    pltpu.async_copy(deg_v, deg_out.at[w], sem_f).wait()


@functools.partial(
    pl.pallas_call,
    out_shape=(jax.ShapeDtypeStruct((E,), jnp.int32),
               jax.ShapeDtypeStruct((E,), jnp.int32)),
)
def _split_kernel(ei_ref, src_ref, dst_ref):
    src_ref[...] = ei_ref[0]
    dst_ref[...] = ei_ref[1]


@functools.partial(
    pl.pallas_call,
    grid=(N_PAD // BLK,),
    in_specs=[
        pl.BlockSpec((BLK, F_IN), lambda i: (i, 0)),
        pl.BlockSpec((F_IN, F_OUT), lambda i: (0, 0)),
        pl.BlockSpec((NW, BLK), lambda i: (0, i)),
    ],
    out_specs=pl.BlockSpec((BLK, F_OUT), lambda i: (i, 0)),
    out_shape=jax.ShapeDtypeStruct((N_PAD, F_OUT), jnp.float32),
)
def _h2_kernel(x_ref, w_ref, deg_ref, h2_ref):
    deg = 1.0 + jnp.sum(deg_ref[...], axis=0)
    dinv = lax.rsqrt(deg)
    h = jnp.dot(x_ref[...], w_ref[...], preferred_element_type=jnp.float32,
                precision=lax.Precision.HIGHEST)
    h2_ref[...] = h * dinv[:, None]


SW = 80
CPB = KC // SW
NBLK = E_PER_W // KC
NCHUNK = E_PER_W // SW
RSLOTS = 4


@functools.partial(
    pl.kernel,
    out_type=jax.ShapeDtypeStruct((NC, N_PAD, F_OUT), jnp.float32),
    mesh=_mesh,
    scratch_types=[
        pltpu.VMEM_SHARED((N_PAD, F_OUT), jnp.float32),
        pltpu.VMEM((SW, F_OUT), jnp.float32),
        pltpu.VMEM((SW, F_OUT), jnp.float32),
        pltpu.VMEM((SW, F_OUT), jnp.float32),
        pltpu.VMEM((SW, F_OUT), jnp.float32),
        pltpu.VMEM((KC,), jnp.int32),
        pltpu.VMEM((KC,), jnp.int32),
        pltpu.VMEM((KC,), jnp.int32),
        pltpu.VMEM((KC,), jnp.int32),
        pltpu.VMEM((KC,), jnp.float32),
        pltpu.VMEM((KC,), jnp.float32),
        pltpu.SemaphoreType.DMA,
        pltpu.SemaphoreType.DMA,
        pltpu.SemaphoreType.DMA,
        pltpu.SemaphoreType.DMA,
        pltpu.SemaphoreType.DMA,
        pltpu.SemaphoreType.DMA,
        pltpu.SemaphoreType.DMA,
        pltpu.SemaphoreType.DMA,
        pltpu.SemaphoreType.DMA,
        pltpu.SemaphoreType.DMA,
    ],
    compiler_params=_sc_params,
)
def _agg_kernel(h2_hbm, src_hbm, dst_hbm, sig_hbm, out_hbm,
                acc_sh, rows0, rows1, rows2, rows3,
                src0, src1, dst0, dst1, sig0, sig1,
                sem_i0, sem_i1,
                sem_g0, sem_g1, sem_g2, sem_g3,
                sem_s0, sem_s1, sem_s2, sem_s3):
    c = lax.axis_index("c")
    s = lax.axis_index("s")
    w = c * NS + s
    base = w * E_PER_W
    row0_ = s * ROWS_PER_TILE

    rows = (rows0, rows1, rows2, rows3)
    src = (src0, src1)
    dst = (dst0, dst1)
    sig = (sig0, sig1)
    sem_i = (sem_i0, sem_i1)
    sem_g = (sem_g0, sem_g1, sem_g2, sem_g3)
    sem_s = (sem_s0, sem_s1, sem_s2, sem_s3)

    def issue_idx(b, isl):
        off = base + b * KC
        pltpu.async_copy(src_hbm.at[pl.ds(off, KC)], src[isl], sem_i[isl])
        pltpu.async_copy(dst_hbm.at[pl.ds(off, KC)], dst[isl], sem_i[isl])
        pltpu.async_copy(sig_hbm.at[pl.ds(off, KC)], sig[isl], sem_i[isl])

    def wait_idx(isl):
        pltpu.make_async_copy(src_hbm.at[pl.ds(0, KC)], src[isl],
                              sem_i[isl]).wait()
        pltpu.make_async_copy(dst_hbm.at[pl.ds(0, KC)], dst[isl],
                              sem_i[isl]).wait()
        pltpu.make_async_copy(sig_hbm.at[pl.ds(0, KC)], sig[isl],
                              sem_i[isl]).wait()

    def issue_gather(rsl, isl, q):
        pltpu.async_copy(h2_hbm.at[src[isl].at[pl.ds(SW * q, SW)]],
                         rows[rsl], sem_g[rsl])

    def wait_gather(rsl, isl, q):
        pltpu.make_async_copy(h2_hbm.at[src[isl].at[pl.ds(SW * q, SW)]],
                              rows[rsl], sem_g[rsl]).wait()

    def issue_scatter(rsl, isl, q):
        pltpu.async_copy(rows[rsl], acc_sh.at[dst[isl].at[pl.ds(SW * q, SW)]],
                         sem_s[rsl], add=True)

    def wait_scatter(rsl, isl, q):
        pltpu.make_async_copy(rows[rsl], acc_sh.at[dst[isl].at[pl.ds(SW * q, SW)]],
                              sem_s[rsl]).wait()

    def scale(rsl, isl, q):
        rv = rows[rsl]
        sv_ref = sig[isl]

        @pl.loop(0, SW, unroll=4)
        def _(k):
            kv = jnp.broadcast_to(SW * q + k, (L,)).astype(jnp.int32)
            sv = plsc.load_gather(sv_ref, [kv])
            for ccol in range(F_OUT // L):
                slc = pl.ds(ccol * L, L)
                rv[k, slc] = rv[k, slc] * sv

    def chunk(q, rsl, isl, q1, rsl1, isl1, *, idx_wait=False, drain=True,
              gissue=True, gissue_pred=None, idx_issue_b=None, idx_issue_sl=0):
        if drain:
            wait_scatter(rsl1, isl1, q1)
        if gissue:
            def _prefetch():
                if idx_wait:
                    wait_idx(isl1)
                issue_gather(rsl1, isl1, q1)
            if gissue_pred is None:
                _prefetch()
            else:
                pl.when(gissue_pred)(_prefetch)
        wait_gather(rsl, isl, q)
        scale(rsl, isl, q)
        issue_scatter(rsl, isl, q)
        if idx_issue_b is not None:
            @pl.when(idx_issue_b < NBLK)
            def _():
                issue_idx(idx_issue_b, idx_issue_sl)

    issue_idx(0, 0)
    issue_idx(1, 1)

    z16 = jnp.zeros((L,), jnp.float32)

    @pl.loop(0, SW, unroll=2)
    def _(k):
        for ccol in range(F_OUT // L):
            rows0[k, pl.ds(ccol * L, L)] = z16

    for tt in range(ROWS_PER_TILE // SW):
        pltpu.sync_copy(rows0, acc_sh.at[pl.ds(row0_ + tt * SW, SW)])
    plsc.subcore_barrier()

    wait_idx(0)
    issue_gather(0, 0, 0)

    chunk(0, 0, 0, 1, 1, 0, drain=False)
    chunk(1, 1, 0, 2, 2, 0, drain=False)
    chunk(2, 2, 0, 3, 3, 0, drain=False)
    chunk(3, 3, 0, 4, 0, 0)
    chunk(4, 0, 0, 0, 1, 1, idx_wait=True)

    NG = (NBLK - 1) // 4

    @pl.loop(0, NG)
    def _(g):
        for j in range(20):
            q = j % 5
            blk_rel = 1 + j // 5
            isl = blk_rel % 2
            rsl = (5 + j) % 4
            q1 = (j + 1) % 5
            isl1 = (1 + (j + 1) // 5) % 2
            rsl1 = (6 + j) % 4
            b_tr = 4 * g + blk_rel
            kw = {}
            if q == 4:
                kw["idx_wait"] = True
            if j == 19:
                kw["gissue_pred"] = g < NG - 1
            if q == 2:
                kw["idx_issue_b"] = b_tr + 1
                kw["idx_issue_sl"] = (blk_rel + 1) % 2
            chunk(q, rsl, isl, q1, rsl1, isl1, **kw)

    wait_scatter(2, 0, 2)
    wait_scatter(3, 0, 3)
    wait_scatter(0, 0, 4)

    plsc.subcore_barrier()
    pltpu.async_copy(acc_sh.at[pl.ds(row0_, ROWS_PER_TILE)],
                     out_hbm.at[c, pl.ds(row0_, ROWS_PER_TILE)], sem_g0).wait()


@functools.partial(
    pl.pallas_call,
    grid=(N_PAD // BLK,),
    in_specs=[
        pl.BlockSpec((1, BLK, F_OUT), lambda i: (0, i, 0)),
        pl.BlockSpec((1, BLK, F_OUT), lambda i: (1, i, 0)),
        pl.BlockSpec((BLK, F_OUT), lambda i: (i, 0)),
        pl.BlockSpec((NW, BLK), lambda i: (0, i)),
        pl.BlockSpec((1, F_OUT), lambda i: (0, 0)),
    ],
    out_specs=pl.BlockSpec((BLK, F_OUT), lambda i: (i, 0)),
    out_shape=jax.ShapeDtypeStruct((N_PAD, F_OUT), jnp.float32),
)
def _out_kernel(p0_ref, p1_ref, h2_ref, deg_ref, b_ref, o_ref):
    deg = 1.0 + jnp.sum(deg_ref[...], axis=0)
    dinv = lax.rsqrt(deg)
    o_ref[...] = (p0_ref[0] + p1_ref[0] + h2_ref[...]) * dinv[:, None] + b_ref[...]


def kernel(x, edge_index, edge_weight, W, b):
    assert x.shape == (N, F_IN) and edge_index.shape == (2, E)
    src, dst = _split_kernel(edge_index)
    x_pad = jnp.pad(x, ((0, N_PAD - N), (0, 0)))

    deg_parts, sig = _deg_kernel(edge_weight, dst)
    h2 = _h2_kernel(x_pad, W, deg_parts)
    parts = _agg_kernel(h2, src, dst, sig)
    out_pad = _out_kernel(parts, parts, h2, deg_parts, b.reshape(1, F_OUT))
    return out_pad[:N]


# speedup vs baseline: 41.4290x; 1.0663x over previous
import dataclasses
import functools

import jax
import jax.numpy as jnp
from jax import lax
from jax.experimental import pallas as pl
from jax.experimental.pallas import tpu as pltpu
from jax.experimental.pallas import tpu_sc as plsc

N = 10000
E = 320000
F_IN = 128
F_OUT = 128

NC = 2
NS = 16
NW = NC * NS
L = 16

N_PAD = 10240
E_PER_W = E // NW
KA = 2000
KC = 400
ROWS_PER_TILE = N_PAD // NS
BLK = 2048

_mesh = plsc.VectorSubcoreMesh(core_axis_name="c", subcore_axis_name="s")

_sc_params = pltpu.CompilerParams()
if "needs_layout_passes" in pltpu.CompilerParams.__dataclass_fields__:
    _sc_params = dataclasses.replace(_sc_params, needs_layout_passes=False)


@functools.partial(
    pl.kernel,
    out_type=jax.ShapeDtypeStruct((NW, N_PAD), jnp.float32),
    mesh=_mesh,
    scratch_types=[
        pltpu.VMEM((N_PAD,), jnp.float32),
        pltpu.VMEM((KA,), jnp.float32),
        pltpu.VMEM((KA,), jnp.float32),
        pltpu.VMEM((KA,), jnp.int32),
        pltpu.VMEM((KA,), jnp.int32),
        pltpu.SemaphoreType.DMA,
        pltpu.SemaphoreType.DMA,
        pltpu.SemaphoreType.DMA,
    ],
    compiler_params=_sc_params,
)
def _deg_kernel(sig_hbm, dst_hbm, deg_out, deg_v,
                sg0, sg1, dst0, dst1, sem_a0, sem_a1, sem_f):
    w = lax.axis_index("c") * NS + lax.axis_index("s")
    base = w * E_PER_W
    zero16 = jnp.zeros((L,), jnp.float32)

    sg = (sg0, sg1)
    dst = (dst0, dst1)
    sem_a = (sem_a0, sem_a1)
    nblk = E_PER_W // KA

    def issue_in(i):
        sl = i % 2
        off = base + i * KA
        pltpu.async_copy(sig_hbm.at[pl.ds(off, KA)], sg[sl], sem_a[sl])
        pltpu.async_copy(dst_hbm.at[pl.ds(off, KA)], dst[sl], sem_a[sl])

    issue_in(0)
    issue_in(1)

    @pl.loop(0, N_PAD, step=L, unroll=4)
    def _(i):
        deg_v[pl.ds(i, L)] = zero16

    for i in range(nblk):
        sl = i % 2
        pltpu.make_async_copy(sig_hbm.at[pl.ds(0, KA)], sg[sl], sem_a[sl]).wait()
        pltpu.make_async_copy(dst_hbm.at[pl.ds(0, KA)], dst[sl], sem_a[sl]).wait()

        @pl.loop(0, KA, step=L, unroll=2)
        def _(j):
            s = sg[sl][pl.ds(j, L)]
            di = dst[sl][pl.ds(j, L)]
            plsc.addupdate_scatter(deg_v, [di], s)

        if i + 2 < nblk:
            issue_in(i + 2)

    pltpu.async_copy(deg_v, deg_out.at[w], sem_f).wait()


@functools.partial(
    pl.pallas_call,
    out_shape=(jax.ShapeDtypeStruct((E,), jnp.int32),
               jax.ShapeDtypeStruct((E,), jnp.int32),
               jax.ShapeDtypeStruct((E,), jnp.float32)),
)
def _split_kernel(ei_ref, ew_ref, src_ref, dst_ref, sig_ref):
    src_ref[...] = ei_ref[0]
    dst_ref[...] = ei_ref[1]
    sig_ref[...] = jax.nn.sigmoid(ew_ref[...])


@functools.partial(
    pl.pallas_call,
    grid=(N_PAD // BLK,),
    in_specs=[
        pl.BlockSpec((BLK, F_IN), lambda i: (i, 0)),
        pl.BlockSpec((F_IN, F_OUT), lambda i: (0, 0)),
        pl.BlockSpec((NW, BLK), lambda i: (0, i)),
    ],
    out_specs=pl.BlockSpec((BLK, F_OUT), lambda i: (i, 0)),
    out_shape=jax.ShapeDtypeStruct((N_PAD, F_OUT), jnp.float32),
)
def _h2_kernel(x_ref, w_ref, deg_ref, h2_ref):
    deg = 1.0 + jnp.sum(deg_ref[...], axis=0)
    dinv = lax.rsqrt(deg)
    h = jnp.dot(x_ref[...], w_ref[...], preferred_element_type=jnp.float32,
                precision=lax.Precision.HIGHEST)
    h2_ref[...] = h * dinv[:, None]


SW = 80
CPB = KC // SW
NBLK = E_PER_W // KC
NCHUNK = E_PER_W // SW
RSLOTS = 4


@functools.partial(
    pl.kernel,
    out_type=jax.ShapeDtypeStruct((NC, N_PAD, F_OUT), jnp.float32),
    mesh=_mesh,
    scratch_types=[
        pltpu.VMEM_SHARED((N_PAD, F_OUT), jnp.float32),
        pltpu.VMEM((SW, F_OUT), jnp.float32),
        pltpu.VMEM((SW, F_OUT), jnp.float32),
        pltpu.VMEM((SW, F_OUT), jnp.float32),
        pltpu.VMEM((SW, F_OUT), jnp.float32),
        pltpu.VMEM((KC,), jnp.int32),
        pltpu.VMEM((KC,), jnp.int32),
        pltpu.VMEM((KC,), jnp.int32),
        pltpu.VMEM((KC,), jnp.int32),
        pltpu.VMEM((KC,), jnp.float32),
        pltpu.VMEM((KC,), jnp.float32),
        pltpu.SemaphoreType.DMA,
        pltpu.SemaphoreType.DMA,
        pltpu.SemaphoreType.DMA,
        pltpu.SemaphoreType.DMA,
        pltpu.SemaphoreType.DMA,
        pltpu.SemaphoreType.DMA,
        pltpu.SemaphoreType.DMA,
        pltpu.SemaphoreType.DMA,
        pltpu.SemaphoreType.DMA,
        pltpu.SemaphoreType.DMA,
    ],
    compiler_params=_sc_params,
)
def _agg_kernel(h2_hbm, src_hbm, dst_hbm, sig_hbm, out_hbm,
                acc_sh, rows0, rows1, rows2, rows3,
                src0, src1, dst0, dst1, sig0, sig1,
                sem_i0, sem_i1,
                sem_g0, sem_g1, sem_g2, sem_g3,
                sem_s0, sem_s1, sem_s2, sem_s3):
    c = lax.axis_index("c")
    s = lax.axis_index("s")
    w = c * NS + s
    base = w * E_PER_W
    row0_ = s * ROWS_PER_TILE

    rows = (rows0, rows1, rows2, rows3)
    src = (src0, src1)
    dst = (dst0, dst1)
    sig = (sig0, sig1)
    sem_i = (sem_i0, sem_i1)
    sem_g = (sem_g0, sem_g1, sem_g2, sem_g3)
    sem_s = (sem_s0, sem_s1, sem_s2, sem_s3)

    def issue_idx(b, isl):
        off = base + b * KC
        pltpu.async_copy(src_hbm.at[pl.ds(off, KC)], src[isl], sem_i[isl])
        pltpu.async_copy(dst_hbm.at[pl.ds(off, KC)], dst[isl], sem_i[isl])
        pltpu.async_copy(sig_hbm.at[pl.ds(off, KC)], sig[isl], sem_i[isl])

    def wait_idx(isl):
        pltpu.make_async_copy(src_hbm.at[pl.ds(0, KC)], src[isl],
                              sem_i[isl]).wait()
        pltpu.make_async_copy(dst_hbm.at[pl.ds(0, KC)], dst[isl],
                              sem_i[isl]).wait()
        pltpu.make_async_copy(sig_hbm.at[pl.ds(0, KC)], sig[isl],
                              sem_i[isl]).wait()

    def issue_gather(rsl, isl, q):
        pltpu.async_copy(h2_hbm.at[src[isl].at[pl.ds(SW * q, SW)]],
                         rows[rsl], sem_g[rsl])

    def wait_gather(rsl, isl, q):
        pltpu.make_async_copy(h2_hbm.at[src[isl].at[pl.ds(SW * q, SW)]],
                              rows[rsl], sem_g[rsl]).wait()

    def issue_scatter(rsl, isl, q):
        pltpu.async_copy(rows[rsl], acc_sh.at[dst[isl].at[pl.ds(SW * q, SW)]],
                         sem_s[rsl], add=True)

    def wait_scatter(rsl, isl, q):
        pltpu.make_async_copy(rows[rsl], acc_sh.at[dst[isl].at[pl.ds(SW * q, SW)]],
                              sem_s[rsl]).wait()

    def scale(rsl, isl, q):
        rv = rows[rsl]
        sv_ref = sig[isl]

        @pl.loop(0, SW, unroll=4)
        def _(k):
            kv = jnp.broadcast_to(SW * q + k, (L,)).astype(jnp.int32)
            sv = plsc.load_gather(sv_ref, [kv])
            for ccol in range(F_OUT // L):
                slc = pl.ds(ccol * L, L)
                rv[k, slc] = rv[k, slc] * sv

    def chunk(q, rsl, isl, q1, rsl1, isl1, *, idx_wait=False, drain=True,
              gissue=True, gissue_pred=None, idx_issue_b=None, idx_issue_sl=0):
        if drain:
            wait_scatter(rsl1, isl1, q1)
        if gissue:
            def _prefetch():
                if idx_wait:
                    wait_idx(isl1)
                issue_gather(rsl1, isl1, q1)
            if gissue_pred is None:
                _prefetch()
            else:
                pl.when(gissue_pred)(_prefetch)
        wait_gather(rsl, isl, q)
        scale(rsl, isl, q)
        issue_scatter(rsl, isl, q)
        if idx_issue_b is not None:
            @pl.when(idx_issue_b < NBLK)
            def _():
                issue_idx(idx_issue_b, idx_issue_sl)

    issue_idx(0, 0)
    issue_idx(1, 1)

    z16 = jnp.zeros((L,), jnp.float32)

    @pl.loop(0, SW, unroll=2)
    def _(k):
        for ccol in range(F_OUT // L):
            rows0[k, pl.ds(ccol * L, L)] = z16

    for tt in range(ROWS_PER_TILE // SW):
        pltpu.sync_copy(rows0, acc_sh.at[pl.ds(row0_ + tt * SW, SW)])
    plsc.subcore_barrier()

    wait_idx(0)
    issue_gather(0, 0, 0)

    chunk(0, 0, 0, 1, 1, 0, drain=False)
    chunk(1, 1, 0, 2, 2, 0, drain=False)
    chunk(2, 2, 0, 3, 3, 0, drain=False)
    chunk(3, 3, 0, 4, 0, 0)
    chunk(4, 0, 0, 0, 1, 1, idx_wait=True)

    NG = (NBLK - 1) // 4

    @pl.loop(0, NG)
    def _(g):
        for j in range(20):
            q = j % 5
            blk_rel = 1 + j // 5
            isl = blk_rel % 2
            rsl = (5 + j) % 4
            q1 = (j + 1) % 5
            isl1 = (1 + (j + 1) // 5) % 2
            rsl1 = (6 + j) % 4
            b_tr = 4 * g + blk_rel
            kw = {}
            if q == 4:
                kw["idx_wait"] = True
            if j == 19:
                kw["gissue_pred"] = g < NG - 1
            if q == 2:
                kw["idx_issue_b"] = b_tr + 1
                kw["idx_issue_sl"] = (blk_rel + 1) % 2
            chunk(q, rsl, isl, q1, rsl1, isl1, **kw)

    wait_scatter(2, 0, 2)
    wait_scatter(3, 0, 3)
    wait_scatter(0, 0, 4)

    plsc.subcore_barrier()
    pltpu.async_copy(acc_sh.at[pl.ds(row0_, ROWS_PER_TILE)],
                     out_hbm.at[c, pl.ds(row0_, ROWS_PER_TILE)], sem_g0).wait()


@functools.partial(
    pl.pallas_call,
    grid=(N_PAD // BLK,),
    in_specs=[
        pl.BlockSpec((1, BLK, F_OUT), lambda i: (0, i, 0)),
        pl.BlockSpec((1, BLK, F_OUT), lambda i: (1, i, 0)),
        pl.BlockSpec((BLK, F_OUT), lambda i: (i, 0)),
        pl.BlockSpec((NW, BLK), lambda i: (0, i)),
        pl.BlockSpec((1, F_OUT), lambda i: (0, 0)),
    ],
    out_specs=pl.BlockSpec((BLK, F_OUT), lambda i: (i, 0)),
    out_shape=jax.ShapeDtypeStruct((N_PAD, F_OUT), jnp.float32),
)
def _out_kernel(p0_ref, p1_ref, h2_ref, deg_ref, b_ref, o_ref):
    deg = 1.0 + jnp.sum(deg_ref[...], axis=0)
    dinv = lax.rsqrt(deg)
    o_ref[...] = (p0_ref[0] + p1_ref[0] + h2_ref[...]) * dinv[:, None] + b_ref[...]


def kernel(x, edge_index, edge_weight, W, b):
    assert x.shape == (N, F_IN) and edge_index.shape == (2, E)
    src, dst, sig = _split_kernel(edge_index, edge_weight)
    x_pad = jnp.pad(x, ((0, N_PAD - N), (0, 0)))

    deg_parts = _deg_kernel(sig, dst)
    h2 = _h2_kernel(x_pad, W, deg_parts)
    parts = _agg_kernel(h2, src, dst, sig)
    out_pad = _out_kernel(parts, parts, h2, deg_parts, b.reshape(1, F_OUT))
    return out_pad[:N]
